# baseline (speedup 1.0000x reference)
import jax
import jax.numpy as jnp
from jax.experimental import pallas as pl
from jax.experimental.pallas import tpu as pltpu

ALPHA_ = 0.5
HB = 16
NSPLIT = 4


def _midmax_body(*refs):
    x_refs = refs[:NSPLIT]
    o_ref = refs[NSPLIT]
    for i, x_ref in enumerate(x_refs):
        wb = x_ref.shape[2]
        blk = x_ref[0].reshape(HB, 2, wb, 64)
        h0 = blk[:, 0]
        h1 = blk[:, 1]
        vmin = jnp.minimum(h0, h1)
        vmax = jnp.maximum(h0, h1)
        vmin4 = vmin.reshape(HB, wb // 2, 2, 64)
        vmax4 = vmax.reshape(HB, wb // 2, 2, 64)
        m1 = vmin4[:, :, 0, :]
        m2 = vmin4[:, :, 1, :]
        M1 = vmax4[:, :, 0, :]
        M2 = vmax4[:, :, 1, :]
        max4 = jnp.maximum(M1, M2)
        sec = jnp.minimum(jnp.maximum(m1, m2), jnp.minimum(M1, M2))
        res = ALPHA_ * max4 + (1.0 - ALPHA_) * jnp.maximum(sec, 0.0)
        o_ref[0, :, i * (wb // 2):(i + 1) * (wb // 2), :] = res


def kernel(x):
    B, H, W, C = x.shape
    Ho, Wo = H // 2, W // 2
    wb = W // NSPLIT
    grid = (B, Ho // HB)
    in_specs = [
        pl.BlockSpec((1, 2 * HB, wb, C),
                     lambda b, h, i=i: (b, h, i, 0))
        for i in range(NSPLIT)
    ]
    out = pl.pallas_call(
        _midmax_body,
        grid=grid,
        in_specs=in_specs,
        out_specs=pl.BlockSpec((1, HB, Wo, C),
                               lambda b, h: (b, h, 0, 0)),
        out_shape=jax.ShapeDtypeStruct((B, Ho, Wo, C), x.dtype),
        compiler_params=pltpu.CompilerParams(
            dimension_semantics=("parallel", "arbitrary")),
    )(*[x] * NSPLIT)
    return out


# speedup vs baseline: 1.2453x; 1.2453x over previous
import jax
import jax.numpy as jnp
from jax.experimental import pallas as pl
from jax.experimental.pallas import tpu as pltpu

ALPHA_ = 0.5
HB = 16


def _midmax_body(x_ref, o_ref):
    v = x_ref[0].reshape(HB, 2, 128, 128)
    h0 = v[:, 0]
    h1 = v[:, 1]
    vmin = jnp.minimum(h0, h1)
    vmax = jnp.maximum(h0, h1)
    m1 = vmin[:, :, :64]
    m2 = vmin[:, :, 64:]
    M1 = vmax[:, :, :64]
    M2 = vmax[:, :, 64:]
    max4 = jnp.maximum(M1, M2)
    sec = jnp.minimum(jnp.maximum(m1, m2), jnp.minimum(M1, M2))
    res = ALPHA_ * max4 + (1.0 - ALPHA_) * jnp.maximum(sec, 0.0)
    r4 = res.reshape(HB, 64, 2, 64)
    o_ref[0] = jnp.concatenate([r4[:, :, 0, :], r4[:, :, 1, :]], axis=-1)


def kernel(x):
    B, H, W, C = x.shape
    Ho, Wo = H // 2, W // 2
    xr = x.reshape(B, H, (W * C) // 128, 128)
    grid = (B, Ho // HB)
    out5 = pl.pallas_call(
        _midmax_body,
        grid=grid,
        in_specs=[pl.BlockSpec((1, 2 * HB, (W * C) // 128, 128),
                               lambda b, h: (b, h, 0, 0))],
        out_specs=pl.BlockSpec((1, HB, (Wo * C) // 128, 128),
                               lambda b, h: (b, h, 0, 0)),
        out_shape=jax.ShapeDtypeStruct((B, Ho, (Wo * C) // 128, 128), x.dtype),
        compiler_params=pltpu.CompilerParams(
            dimension_semantics=("parallel", "arbitrary")),
    )(xr)
    return out5.reshape(B, Ho, Wo, C)
